# baseline (speedup 1.0000x reference)
import functools

import jax
import jax.numpy as jnp
from jax import lax
from jax.experimental import pallas as pl
from jax.experimental.pallas import tpu as pltpu
from jax.experimental.pallas import tpu_sc as plsc

_NC = 19
_PAD = 320
_BLK = 256
_NB = 64
_SCC = 192
_CH = _SCC // 2
_HWB = 16384
_TCB = 32
_KB = 2048


def _lane_transpose_reduce(ref, iota16, off):
    zero = jnp.zeros((16,), jnp.float32)
    s0 = zero
    s1 = zero
    for r in range(16):
        g0 = plsc.load_gather(ref, [iota16 * 16 + (r + off)])
        g1 = plsc.load_gather(ref, [(iota16 + 16) * 16 + (r + off)])
        s0 = s0 + g0
        s1 = s1 + g1
    return s0, s1


_mesh = plsc.VectorSubcoreMesh(core_axis_name="c", subcore_axis_name="s")


@functools.partial(
    pl.kernel,
    mesh=_mesh,
    compiler_params=pltpu.CompilerParams(needs_layout_passes=False),
    out_type=(
        jax.ShapeDtypeStruct((2, 16, _CH * 32), jnp.float32),
        jax.ShapeDtypeStruct((2, 16, _HWB), jnp.float32),
    ),
    scratch_types=[
        pltpu.VMEM((2, _CH, _BLK), jnp.float32),
        pltpu.VMEM((2, _BLK), jnp.int32),
        pltpu.VMEM((_CH * _PAD + 512,), jnp.float32),
        pltpu.VMEM((_HWB,), jnp.float32),
        pltpu.VMEM((_CH * 32,), jnp.float32),
        pltpu.SemaphoreType.DMA,
        pltpu.SemaphoreType.DMA,
        pltpu.SemaphoreType.DMA,
        pltpu.SemaphoreType.DMA,
    ],
)
def _sc_pass(f_hbm, y_hbm, sums_out, psq_out,
             buf, labbuf, acc, psq, stage, sd0, sd1, sl0, sl1):
    half = lax.axis_index("c")
    b = lax.axis_index("s")
    c0 = half * _CH
    sems_d = (sd0, sd1)
    sems_l = (sl0, sl1)

    zero = jnp.zeros((16,), jnp.float32)

    def _zbody(i, carry):
        acc[pl.ds(i * 16, 16)] = zero
        return carry

    lax.fori_loop(0, (_CH * _PAD + 512) // 16, _zbody, 0)

    def _data_copy(pb, slot):
        return pltpu.make_async_copy(
            f_hbm.at[b, pl.ds(c0, _CH), pl.ds(pb * _BLK, _BLK)],
            buf.at[slot], sems_d[slot])

    def _lab_copy(pb, slot):
        return pltpu.make_async_copy(
            y_hbm.at[b, pl.ds(pb * _BLK, _BLK)],
            labbuf.at[slot], sems_l[slot])

    _data_copy(0, 0).start()
    _lab_copy(0, 0).start()

    iota16 = lax.iota(jnp.int32, 16)

    def _outer(g2, carry):
        for s in range(2):
            pb = g2 * 2 + s

            @pl.when(pb + 1 < _NB)
            def _start_next():
                _data_copy(pb + 1, 1 - s).start()
                _lab_copy(pb + 1, 1 - s).start()

            _data_copy(pb, s).wait()
            _lab_copy(pb, s).wait()

            idxs = [labbuf[s, pl.ds(j * 16, 16)] * 16 + iota16
                    for j in range(16)]

            def _cbody(c, ps, s=s, idxs=idxs):
                out = list(ps)
                for j in range(16):
                    v = buf[s, c, pl.ds(j * 16, 16)]
                    plsc.addupdate_scatter(acc.at[pl.ds(c * _PAD, _PAD)],
                                           [idxs[j]], v)
                    out[j] = out[j] + v * v
                return tuple(out)

            ps = plsc.parallel_loop(0, _CH, unroll=4,
                                    carry=(zero,) * 16)(_cbody)
            for j in range(16):
                psq[pl.ds(pb * _BLK + j * 16, 16)] = ps[j]
        return carry

    lax.fori_loop(0, _NB // 2, _outer, 0)

    for c in range(_CH):
        s0, s1 = _lane_transpose_reduce(acc, iota16, c * _PAD)
        stage[pl.ds(c * 32, 16)] = s0
        stage[pl.ds(c * 32 + 16, 16)] = s1

    pltpu.sync_copy(psq, psq_out.at[half, b])
    pltpu.sync_copy(stage, sums_out.at[half, b])


def _tc_chan_body(f_ref, y_ref, sums_ref, psq_ref):
    b = pl.program_id(0)
    k = pl.program_id(1)

    @pl.when(jnp.logical_and(b == 0, k == 0))
    def _init_sums():
        sums_ref[...] = jnp.zeros_like(sums_ref)

    f = f_ref[0]
    lab = y_ref[0, 0]
    ncheff = f.shape[0]
    classes = jax.lax.broadcasted_iota(jnp.int32, (_KB, _NC), 1)
    onehot = (lab[:, None] == classes).astype(jnp.float32)

    sums_ref[...] += jnp.dot(f, onehot,
                             preferred_element_type=jnp.float32)
    ones_row = jnp.ones((1, ncheff), jnp.float32)
    psq_ref[0] = jnp.dot(ones_row, f * f,
                         preferred_element_type=jnp.float32)


def _tc_finish_body(p0_ref, p1_ref, pt_ref, y_ref, nsum_ref, cnt_ref):
    b = pl.program_id(0)
    k = pl.program_id(1)

    @pl.when(jnp.logical_and(b == 0, k == 0))
    def _init():
        nsum_ref[...] = jnp.zeros_like(nsum_ref)
        cnt_ref[...] = jnp.zeros_like(cnt_ref)

    norms = jnp.sqrt(p0_ref[0] + p1_ref[0] + pt_ref[0])
    lab = y_ref[0]
    classes = jax.lax.broadcasted_iota(jnp.int32, (_NC, _KB), 0)
    oh19 = (lab == classes).astype(jnp.float32)

    nsum_ref[...] += lax.dot_general(
        norms, oh19, (((1,), (1,)), ((), ())),
        preferred_element_type=jnp.float32)
    cnt_ref[...] += lax.dot_general(
        jnp.ones((1, _KB), jnp.float32), oh19, (((1,), (1,)), ((), ())),
        preferred_element_type=jnp.float32)


def kernel(feats, y_down):
    B, C, H, W = feats.shape
    HW = H * W
    f3 = feats.reshape(B, C, HW)
    y2 = y_down.reshape(B, HW)
    y3 = y_down.reshape(B, 1, HW)
    tcc = C - _SCC

    sums_sc, psq_sc = _sc_pass(f3, y2)
    psq_sc2 = psq_sc.reshape(32, 1, HW)

    sums_tc, psq_tc = pl.pallas_call(
        _tc_chan_body,
        grid=(B, HW // _KB),
        in_specs=[
            pl.BlockSpec((1, tcc, _KB), lambda b, k: (b, _SCC // tcc, k)),
            pl.BlockSpec((1, 1, _KB), lambda b, k: (b, 0, k)),
        ],
        out_specs=[
            pl.BlockSpec((tcc, _NC), lambda b, k: (0, 0)),
            pl.BlockSpec((1, 1, _KB), lambda b, k: (b, 0, k)),
        ],
        out_shape=[
            jax.ShapeDtypeStruct((tcc, _NC), jnp.float32),
            jax.ShapeDtypeStruct((B, 1, HW), jnp.float32),
        ],
    )(f3, y3)

    nsum, cnt = pl.pallas_call(
        _tc_finish_body,
        grid=(B, HW // _KB),
        in_specs=[
            pl.BlockSpec((1, 1, _KB), lambda b, k: (b, 0, k)),
            pl.BlockSpec((1, 1, _KB), lambda b, k: (16 + b, 0, k)),
            pl.BlockSpec((1, 1, _KB), lambda b, k: (b, 0, k)),
            pl.BlockSpec((1, 1, _KB), lambda b, k: (b, 0, k)),
        ],
        out_specs=[
            pl.BlockSpec((1, _NC), lambda b, k: (0, 0)),
            pl.BlockSpec((1, _NC), lambda b, k: (0, 0)),
        ],
        out_shape=[
            jax.ShapeDtypeStruct((1, _NC), jnp.float32),
            jax.ShapeDtypeStruct((1, _NC), jnp.float32),
        ],
    )(psq_sc2, psq_sc2, psq_tc, y3)

    sc_part = sums_sc.reshape(2, 16, _CH, 32).sum(axis=1)
    sc_sums = sc_part.reshape(_SCC, 32)[:, :_NC]
    sums = jnp.concatenate([sc_sums, sums_tc], axis=0)
    counts = cnt[0]
    safe = jnp.maximum(counts, 1.0)
    b_c = (sums / safe[None, :]).T
    n_c = nsum[0] / safe
    return b_c, n_c


# speedup vs baseline: 1.0288x; 1.0288x over previous
import functools

import jax
import jax.numpy as jnp
from jax import lax
from jax.experimental import pallas as pl
from jax.experimental.pallas import tpu as pltpu
from jax.experimental.pallas import tpu_sc as plsc

_NC = 19
_PAD = 320
_BLK = 256
_NB = 64
_SCC = 128
_CH = _SCC // 2
_HWB = 16384
_TCB = 32
_KB = 2048


def _lane_transpose_reduce(ref, iota16, off):
    zero = jnp.zeros((16,), jnp.float32)
    s0 = zero
    s1 = zero
    for r in range(16):
        g0 = plsc.load_gather(ref, [iota16 * 16 + (r + off)])
        g1 = plsc.load_gather(ref, [(iota16 + 16) * 16 + (r + off)])
        s0 = s0 + g0
        s1 = s1 + g1
    return s0, s1


_mesh = plsc.VectorSubcoreMesh(core_axis_name="c", subcore_axis_name="s")


@functools.partial(
    pl.kernel,
    mesh=_mesh,
    compiler_params=pltpu.CompilerParams(needs_layout_passes=False),
    out_type=(
        jax.ShapeDtypeStruct((2, 16, _CH * 32), jnp.float32),
        jax.ShapeDtypeStruct((2, 16, _HWB), jnp.float32),
    ),
    scratch_types=[
        pltpu.VMEM((2, _CH, _BLK), jnp.float32),
        pltpu.VMEM((2, _BLK), jnp.int32),
        pltpu.VMEM((_CH * _PAD + 512,), jnp.float32),
        pltpu.VMEM((_HWB,), jnp.float32),
        pltpu.VMEM((_CH * 32,), jnp.float32),
        pltpu.SemaphoreType.DMA,
        pltpu.SemaphoreType.DMA,
        pltpu.SemaphoreType.DMA,
        pltpu.SemaphoreType.DMA,
    ],
)
def _sc_pass(f_hbm, y_hbm, sums_out, psq_out,
             buf, labbuf, acc, psq, stage, sd0, sd1, sl0, sl1):
    half = lax.axis_index("c")
    b = lax.axis_index("s")
    c0 = half * _CH
    sems_d = (sd0, sd1)
    sems_l = (sl0, sl1)

    zero = jnp.zeros((16,), jnp.float32)

    def _zbody(i, carry):
        acc[pl.ds(i * 16, 16)] = zero
        return carry

    lax.fori_loop(0, (_CH * _PAD + 512) // 16, _zbody, 0)

    def _data_copy(pb, slot):
        return pltpu.make_async_copy(
            f_hbm.at[b, pl.ds(c0, _CH), pl.ds(pb * _BLK, _BLK)],
            buf.at[slot], sems_d[slot])

    def _lab_copy(pb, slot):
        return pltpu.make_async_copy(
            y_hbm.at[b, pl.ds(pb * _BLK, _BLK)],
            labbuf.at[slot], sems_l[slot])

    _data_copy(0, 0).start()
    _lab_copy(0, 0).start()

    iota16 = lax.iota(jnp.int32, 16)

    def _outer(g2, carry):
        for s in range(2):
            pb = g2 * 2 + s

            @pl.when(pb + 1 < _NB)
            def _start_next():
                _data_copy(pb + 1, 1 - s).start()
                _lab_copy(pb + 1, 1 - s).start()

            _data_copy(pb, s).wait()
            _lab_copy(pb, s).wait()

            idxs = [labbuf[s, pl.ds(j * 16, 16)] * 16 + iota16
                    for j in range(16)]

            def _cbody(c, ps, s=s, idxs=idxs):
                out = list(ps)
                for j in range(16):
                    v = buf[s, c, pl.ds(j * 16, 16)]
                    plsc.addupdate_scatter(acc.at[pl.ds(c * _PAD, _PAD)],
                                           [idxs[j]], v)
                    out[j] = out[j] + v * v
                return tuple(out)

            ps = plsc.parallel_loop(0, _CH, unroll=4,
                                    carry=(zero,) * 16)(_cbody)
            for j in range(16):
                psq[pl.ds(pb * _BLK + j * 16, 16)] = ps[j]
        return carry

    lax.fori_loop(0, _NB // 2, _outer, 0)

    for c in range(_CH):
        s0, s1 = _lane_transpose_reduce(acc, iota16, c * _PAD)
        stage[pl.ds(c * 32, 16)] = s0
        stage[pl.ds(c * 32 + 16, 16)] = s1

    pltpu.sync_copy(psq, psq_out.at[half, b])
    pltpu.sync_copy(stage, sums_out.at[half, b])


def _tc_chan_body(f_ref, y_ref, sums_ref, psq_ref):
    b = pl.program_id(0)
    k = pl.program_id(1)

    @pl.when(jnp.logical_and(b == 0, k == 0))
    def _init_sums():
        sums_ref[...] = jnp.zeros_like(sums_ref)

    f = f_ref[0]
    lab = y_ref[0, 0]
    ncheff = f.shape[0]
    classes = jax.lax.broadcasted_iota(jnp.int32, (_KB, _NC), 1)
    onehot = (lab[:, None] == classes).astype(jnp.float32)

    sums_ref[...] += jnp.dot(f, onehot,
                             preferred_element_type=jnp.float32)
    ones_row = jnp.ones((1, ncheff), jnp.float32)
    psq_ref[0] = jnp.dot(ones_row, f * f,
                         preferred_element_type=jnp.float32)


def _tc_finish_body(p0_ref, p1_ref, pt_ref, y_ref, nsum_ref, cnt_ref):
    b = pl.program_id(0)
    k = pl.program_id(1)

    @pl.when(jnp.logical_and(b == 0, k == 0))
    def _init():
        nsum_ref[...] = jnp.zeros_like(nsum_ref)
        cnt_ref[...] = jnp.zeros_like(cnt_ref)

    norms = jnp.sqrt(p0_ref[0] + p1_ref[0] + pt_ref[0])
    lab = y_ref[0]
    classes = jax.lax.broadcasted_iota(jnp.int32, (_NC, _KB), 0)
    oh19 = (lab == classes).astype(jnp.float32)

    nsum_ref[...] += lax.dot_general(
        norms, oh19, (((1,), (1,)), ((), ())),
        preferred_element_type=jnp.float32)
    cnt_ref[...] += lax.dot_general(
        jnp.ones((1, _KB), jnp.float32), oh19, (((1,), (1,)), ((), ())),
        preferred_element_type=jnp.float32)


def kernel(feats, y_down):
    B, C, H, W = feats.shape
    HW = H * W
    f3 = feats.reshape(B, C, HW)
    y2 = y_down.reshape(B, HW)
    y3 = y_down.reshape(B, 1, HW)
    tcc = C - _SCC

    sums_sc, psq_sc = _sc_pass(f3, y2)
    psq_sc2 = psq_sc.reshape(32, 1, HW)

    sums_tc, psq_tc = pl.pallas_call(
        _tc_chan_body,
        grid=(B, HW // _KB),
        in_specs=[
            pl.BlockSpec((1, tcc, _KB), lambda b, k: (b, _SCC // tcc, k)),
            pl.BlockSpec((1, 1, _KB), lambda b, k: (b, 0, k)),
        ],
        out_specs=[
            pl.BlockSpec((tcc, _NC), lambda b, k: (0, 0)),
            pl.BlockSpec((1, 1, _KB), lambda b, k: (b, 0, k)),
        ],
        out_shape=[
            jax.ShapeDtypeStruct((tcc, _NC), jnp.float32),
            jax.ShapeDtypeStruct((B, 1, HW), jnp.float32),
        ],
    )(f3, y3)

    nsum, cnt = pl.pallas_call(
        _tc_finish_body,
        grid=(B, HW // _KB),
        in_specs=[
            pl.BlockSpec((1, 1, _KB), lambda b, k: (b, 0, k)),
            pl.BlockSpec((1, 1, _KB), lambda b, k: (16 + b, 0, k)),
            pl.BlockSpec((1, 1, _KB), lambda b, k: (b, 0, k)),
            pl.BlockSpec((1, 1, _KB), lambda b, k: (b, 0, k)),
        ],
        out_specs=[
            pl.BlockSpec((1, _NC), lambda b, k: (0, 0)),
            pl.BlockSpec((1, _NC), lambda b, k: (0, 0)),
        ],
        out_shape=[
            jax.ShapeDtypeStruct((1, _NC), jnp.float32),
            jax.ShapeDtypeStruct((1, _NC), jnp.float32),
        ],
    )(psq_sc2, psq_sc2, psq_tc, y3)

    sc_part = sums_sc.reshape(2, 16, _CH, 32).sum(axis=1)
    sc_sums = sc_part.reshape(_SCC, 32)[:, :_NC]
    sums = jnp.concatenate([sc_sums, sums_tc], axis=0)
    counts = cnt[0]
    safe = jnp.maximum(counts, 1.0)
    b_c = (sums / safe[None, :]).T
    n_c = nsum[0] / safe
    return b_c, n_c
